# baseline (speedup 1.0000x reference)
import functools

import jax
import jax.numpy as jnp
from jax import lax
from jax.experimental import pallas as pl
from jax.experimental.pallas import tpu as pltpu
from jax.experimental.pallas import tpu_sc as plsc

_REP = 8
_RSTRIDE = 3 * 128 * 32 + 2


def _tap_tables_tc(embed, conv_wt, conv_b2):
    def body(e_ref, w_ref, b_ref, out_ref):
        e = e_ref[...]
        for k in range(3):
            wk = w_ref[k]
            tk = lax.dot_general(
                e, wk,
                dimension_numbers=(((1,), (1,)), ((), ())),
                preferred_element_type=jnp.float32,
            )
            if k == 1:
                tk = tk + b_ref[...]
            out_ref[k] = tk
        return None

    return pl.pallas_call(
        body,
        out_shape=jax.ShapeDtypeStruct((3, 128, 32), jnp.float32),
    )(embed, conv_wt, conv_b2)


def _make_sc_kernel(n_tok):
    n_workers = 32
    tok_w = n_tok // n_workers
    groups = tok_w // 16
    ids_w = tok_w * 10
    tab_w = (_REP - 1) * _RSTRIDE + 3 * 128 * 32
    out_w = tok_w * 33

    mesh = plsc.VectorSubcoreMesh(core_axis_name="c", subcore_axis_name="s")

    @functools.partial(
        pl.kernel,
        out_type=jax.ShapeDtypeStruct((n_tok * 33,), jnp.float32),
        mesh=mesh,
        scratch_types=[
            pltpu.VMEM((ids_w,), jnp.int32),
            pltpu.VMEM((tab_w,), jnp.float32),
            pltpu.VMEM((16 * 33,), jnp.float32),
            pltpu.SemaphoreType.DMA,
        ],
        compiler_params=pltpu.CompilerParams(needs_layout_passes=False),
    )
    def sc_main(ids_hbm, tab_hbm, out_hbm, ids_v, tab_v, outc_v, osem):
        wid = lax.axis_index("s") * 2 + lax.axis_index("c")
        pltpu.sync_copy(ids_hbm.at[pl.ds(wid * ids_w, ids_w)], ids_v)
        pltpu.sync_copy(tab_hbm, tab_v)

        iota = lax.iota(jnp.int32, 16)
        iota10 = iota * 10
        iota33 = iota * 33
        rbase = (iota & 7) * _RSTRIDE

        def group(g, carry):
            cbase = g * (16 * 10)
            c32 = []
            for e in range(10):
                pk = plsc.load_gather(ids_v, [iota10 + (cbase + e)])
                c32.append((pk & 0x7F) * 32 + rbase)
                c32.append(lax.shift_right_logical(pk, 20) + rbase)
            @pl.when(g > 0)
            def _():
                pltpu.make_async_copy(
                    outc_v, out_hbm.at[pl.ds(0, 16 * 33)], osem,
                ).wait()
            for j in range(32):
                acc = None
                for i in range(20):
                    f = 32 * i + j
                    w = f % 20
                    o = f // 20
                    v = plsc.load_gather(tab_v, [c32[w] + (4096 + o)])
                    if w > 0:
                        v = v + plsc.load_gather(tab_v, [c32[w - 1] + o])
                    if w < 19:
                        v = v + plsc.load_gather(
                            tab_v, [c32[w + 1] + (8192 + o)])
                    acc = v if acc is None else jnp.maximum(acc, v)
                plsc.store_scatter(outc_v, [iota33 + j], acc)
            pltpu.async_copy(
                outc_v, out_hbm.at[pl.ds(wid * out_w + g * (16 * 33),
                                         16 * 33)], osem)
            return carry

        lax.fori_loop(0, groups, group, 0)
        pltpu.make_async_copy(
            outc_v, out_hbm.at[pl.ds(0, 16 * 33)], osem,
        ).wait()

    return sc_main


def kernel(char_ids, embed_table, conv_w, conv_b):
    b, s, w = char_ids.shape
    d = embed_table.shape[1]
    n_tok = b * s

    conv_wt = conv_w.astype(jnp.float32).transpose(2, 0, 1)
    conv_b2 = conv_b.astype(jnp.float32).reshape(1, d)

    tab = _tap_tables_tc(embed_table.astype(jnp.float32), conv_wt, conv_b2)
    t_unit = jnp.concatenate([tab.reshape(-1), jnp.zeros((2,), jnp.float32)])
    tab_rep = jnp.tile(t_unit, _REP)[:(_REP - 1) * _RSTRIDE + 3 * 128 * 32]

    ids2 = char_ids.astype(jnp.int32).reshape(n_tok, w // 2, 2)
    ids_flat = (ids2[..., 0] | (ids2[..., 1] << 25)).reshape(n_tok * (w // 2))

    sc_main = _make_sc_kernel(n_tok)
    out_flat = sc_main(ids_flat, tab_rep)
    return out_flat.reshape(n_tok, 33)[:, :d].reshape(b, s, d)


# speedup vs baseline: 1.6565x; 1.6565x over previous
import functools

import jax
import jax.numpy as jnp
from jax import lax
from jax.experimental import pallas as pl
from jax.experimental.pallas import tpu as pltpu
from jax.experimental.pallas import tpu_sc as plsc


def _tap_tables_tc(e_pad, conv_wt, conv_b2):
    def body(e_ref, w_ref, b_ref, out_ref):
        e = e_ref[...]
        for k in range(3):
            wk = w_ref[k]
            tk = lax.dot_general(
                e, wk,
                dimension_numbers=(((1,), (1,)), ((), ())),
                preferred_element_type=jnp.float32,
            )
            if k == 1:
                tk = tk + b_ref[...]
            out_ref[k] = tk
        return None

    return pl.pallas_call(
        body,
        out_shape=jax.ShapeDtypeStruct((3, 256, 32), jnp.float32),
    )(e_pad, conv_wt, conv_b2)


def _make_sc_kernel(n_tok):
    n_workers = 32
    tok_w = n_tok // n_workers
    groups = tok_w // 16
    ids_w = tok_w * 21
    out_w = tok_w * 33

    mesh = plsc.VectorSubcoreMesh(core_axis_name="c", subcore_axis_name="s")

    @functools.partial(
        pl.kernel,
        out_type=jax.ShapeDtypeStruct((n_tok * 33,), jnp.float32),
        mesh=mesh,
        scratch_types=[
            pltpu.VMEM((ids_w,), jnp.int32),
            pltpu.VMEM((3 * 256 * 33,), jnp.float32),
            pltpu.VMEM((out_w,), jnp.float32),
        ],
        compiler_params=pltpu.CompilerParams(needs_layout_passes=False),
    )
    def sc_main(ids_hbm, tab_hbm, out_hbm, ids_v, tab_v, out_v):
        wid = lax.axis_index("s") * 2 + lax.axis_index("c")
        pltpu.sync_copy(ids_hbm.at[pl.ds(wid * ids_w, ids_w)], ids_v)
        pltpu.sync_copy(tab_hbm, tab_v)

        iota = lax.iota(jnp.int32, 16)
        iota21 = iota * 21
        iota33 = iota * 33

        def group(g, carry):
            cbase = g * (16 * 21)
            obase = g * (16 * 33)
            c33 = []
            for e in range(20):
                c = plsc.load_gather(ids_v, [iota21 + (cbase + e)])
                c33.append(c * 33)
            for j in range(32):
                acc = jnp.full((16,), -jnp.inf, jnp.float32)
                for i in range(20):
                    f = 32 * i + j
                    w = f % 20
                    o = f // 20
                    v = plsc.load_gather(tab_v, [c33[w] + (8448 + o)])
                    if w > 0:
                        v = v + plsc.load_gather(tab_v, [c33[w - 1] + o])
                    if w < 19:
                        v = v + plsc.load_gather(
                            tab_v, [c33[w + 1] + (16896 + o)])
                    acc = jnp.maximum(acc, v)
                plsc.store_scatter(out_v, [iota33 + (obase + j)], acc)
            return carry

        lax.fori_loop(0, groups, group, 0)
        pltpu.sync_copy(out_v, out_hbm.at[pl.ds(wid * out_w, out_w)])

    return sc_main


def kernel(char_ids, embed_table, conv_w, conv_b):
    b, s, w = char_ids.shape
    d = embed_table.shape[1]
    n_tok = b * s

    e_pad = jnp.pad(embed_table.astype(jnp.float32), ((0, 128), (0, 0)))
    conv_wt = conv_w.astype(jnp.float32).transpose(2, 0, 1)
    conv_b2 = conv_b.astype(jnp.float32).reshape(1, d)

    tables = _tap_tables_tc(e_pad, conv_wt, conv_b2)
    tab_pad = jnp.pad(tables, ((0, 0), (0, 0), (0, 1)))

    ids = char_ids.astype(jnp.int32).reshape(n_tok, w)
    ids_pad = jnp.pad(ids, ((0, 0), (0, 1)))
    ids_flat = ids_pad.reshape(n_tok * 21)

    sc_main = _make_sc_kernel(n_tok)
    out_flat = sc_main(ids_flat, tab_pad.reshape(3 * 256 * 33))
    return out_flat.reshape(n_tok, 33)[:, :d].reshape(b, s, d)


# speedup vs baseline: 1.6619x; 1.0033x over previous
import functools

import jax
import jax.numpy as jnp
from jax import lax
from jax.experimental import pallas as pl
from jax.experimental.pallas import tpu as pltpu
from jax.experimental.pallas import tpu_sc as plsc


def _tap_tables_tc(e_pad, conv_wt, conv_b2):
    def body(e_ref, w_ref, b_ref, out_ref):
        e = e_ref[...]
        for k in range(3):
            wk = w_ref[k]
            tk = lax.dot_general(
                e, wk,
                dimension_numbers=(((1,), (1,)), ((), ())),
                preferred_element_type=jnp.float32,
            )
            if k == 1:
                tk = tk + b_ref[...]
            out_ref[k] = tk
        return None

    return pl.pallas_call(
        body,
        out_shape=jax.ShapeDtypeStruct((3, 256, 32), jnp.float32),
    )(e_pad, conv_wt, conv_b2)


def _make_sc_kernel(n_tok):
    n_workers = 32
    tok_w = n_tok // n_workers
    groups = tok_w // 16
    ids_w = tok_w * 21
    out_w = tok_w * 33

    mesh = plsc.VectorSubcoreMesh(core_axis_name="c", subcore_axis_name="s")

    @functools.partial(
        pl.kernel,
        out_type=jax.ShapeDtypeStruct((n_tok * 33,), jnp.float32),
        mesh=mesh,
        scratch_types=[
            pltpu.VMEM((ids_w,), jnp.int32),
            pltpu.VMEM((3 * 256 * 33,), jnp.float32),
            pltpu.VMEM((out_w,), jnp.float32),
        ],
        compiler_params=pltpu.CompilerParams(needs_layout_passes=False),
    )
    def sc_main(ids_hbm, tab_hbm, out_hbm, ids_v, tab_v, out_v):
        wid = lax.axis_index("s") * 2 + lax.axis_index("c")
        pltpu.sync_copy(ids_hbm.at[pl.ds(wid * ids_w, ids_w)], ids_v)
        pltpu.sync_copy(tab_hbm, tab_v)

        iota = lax.iota(jnp.int32, 16)
        iota21 = iota * 21
        iota33 = iota * 33

        def group(g, carry):
            cbase = g * (16 * 21)
            obase = g * (16 * 33)
            c33 = []
            for e in range(20):
                c = plsc.load_gather(ids_v, [iota21 + (cbase + e)])
                c33.append(c * 33)
            for j in range(32):
                acc = None
                for i in range(20):
                    f = 32 * i + j
                    w = f % 20
                    o = f // 20
                    v = plsc.load_gather(tab_v, [c33[w] + (8448 + o)])
                    if w > 0:
                        v = v + plsc.load_gather(tab_v, [c33[w - 1] + o])
                    if w < 19:
                        v = v + plsc.load_gather(
                            tab_v, [c33[w + 1] + (16896 + o)])
                    acc = v if acc is None else jnp.maximum(acc, v)
                plsc.store_scatter(out_v, [iota33 + (obase + j)], acc)
            return carry

        lax.fori_loop(0, groups, group, 0)
        pltpu.sync_copy(out_v, out_hbm.at[pl.ds(wid * out_w, out_w)])

    return sc_main


def kernel(char_ids, embed_table, conv_w, conv_b):
    b, s, w = char_ids.shape
    d = embed_table.shape[1]
    n_tok = b * s

    e_pad = jnp.pad(embed_table.astype(jnp.float32), ((0, 128), (0, 0)))
    conv_wt = conv_w.astype(jnp.float32).transpose(2, 0, 1)
    conv_b2 = conv_b.astype(jnp.float32).reshape(1, d)

    tables = _tap_tables_tc(e_pad, conv_wt, conv_b2)
    tab_pad = jnp.pad(tables, ((0, 0), (0, 0), (0, 1)))

    ids = char_ids.astype(jnp.int32).reshape(n_tok, w)
    ids_pad = jnp.pad(ids, ((0, 0), (0, 1)))
    ids_flat = ids_pad.reshape(n_tok * 21)

    sc_main = _make_sc_kernel(n_tok)
    out_flat = sc_main(ids_flat, tab_pad.reshape(3 * 256 * 33))
    return out_flat.reshape(n_tok, 33)[:, :d].reshape(b, s, d)
